# baseline (speedup 1.0000x reference)
import functools

import jax
import jax.numpy as jnp
from jax import lax
from jax.experimental import pallas as pl
from jax.experimental.pallas import tpu as pltpu
from jax.experimental.pallas import tpu_sc as plsc

B, L, V, D = 16384, 50, 1000000, 32
VP = 1000064

NC, NS = 2, 16
NW = NC * NS
BPW = B // NW
JG = BPW // 16
LP = 56

_ND = D // 8
_CH = 249856
_NCH = 4
_TAIL = V - _NCH * _CH
_CHUNKS = [(k * _CH, _CH) for k in range(_NCH)] + [(_NCH * _CH, _TAIL)]
_NBUF = 3


def _tc_project_body(tv_hbm, w_ref, t_ref, buf, tbuf, sems):
    jobs = [(i, off, n) for i in range(_ND) for (off, n) in _CHUNKS]

    def copy_in(slot, job):
        i, off, n = job
        dst = buf.at[slot] if n == _CH else tbuf.at[slot]
        return pltpu.make_async_copy(
            tv_hbm.at[pl.ds(8 * i, 8), pl.ds(off, n)],
            dst,
            sems.at[slot],
        )

    nbuf = _NBUF
    for p in range(min(nbuf - 1, len(jobs))):
        copy_in(p % nbuf, jobs[p]).start()
    for j, job in enumerate(jobs):
        if j + nbuf - 1 < len(jobs):
            copy_in((j + nbuf - 1) % nbuf, jobs[j + nbuf - 1]).start()
        copy_in(j % nbuf, job).wait()
        i, off, n = job
        src = buf[j % nbuf] if n == _CH else tbuf[j % nbuf]
        part = jnp.sum(src * w_ref[pl.ds(8 * i, 8), :], axis=0)
        if i == 0:
            t_ref[pl.ds(off, n)] = part
        else:
            t_ref[pl.ds(off, n)] += part


def _project_table(table, W):
    tv = table.T
    wcol = W.reshape(D, 1)
    t = pl.pallas_call(
        _tc_project_body,
        in_specs=[
            pl.BlockSpec(memory_space=pl.ANY),
            pl.BlockSpec((D, 1), lambda: (0, 0)),
        ],
        out_specs=pl.BlockSpec((VP,), lambda: (0,)),
        out_shape=jax.ShapeDtypeStruct((VP,), jnp.float32),
        scratch_shapes=[
            pltpu.VMEM((_NBUF, 8, _CH), jnp.float32),
            pltpu.VMEM((_NBUF, 8, _TAIL), jnp.float32),
            pltpu.SemaphoreType.DMA((_NBUF,)),
        ],
    )(tv, wcol)
    return t


def _tc_reblock_body(idx_ref, out_ref):
    out_ref[pl.ds(0, L), :] = idx_ref[...]


def _reblock_idx(idx_t):
    return pl.pallas_call(
        _tc_reblock_body,
        grid=(NW,),
        in_specs=[pl.BlockSpec((L, BPW), lambda i: (0, i))],
        out_specs=pl.BlockSpec((LP, BPW), lambda i: (i, 0)),
        out_shape=jax.ShapeDtypeStruct((NW * LP, BPW), jnp.int32),
    )(idx_t)


@functools.partial(
    pl.kernel,
    out_type=jax.ShapeDtypeStruct((B,), jnp.float32),
    mesh=plsc.VectorSubcoreMesh(core_axis_name="c", subcore_axis_name="s"),
    scratch_types=[
        pltpu.VMEM((BPW * L,), jnp.int32),
        pltpu.VMEM((BPW * L,), jnp.float32),
        pltpu.VMEM((BPW,), jnp.float32),
        pltpu.VMEM_SHARED((VP,), jnp.float32),
        pltpu.SemaphoreType.DMA((2,)),
        pltpu.SemaphoreType.DMA,
    ],
)
def _sc_gather_sum(idx_hbm, t_hbm, out_hbm, idx_t, vals_v, acc_v, t_sh,
                   sems, ssem):
    sid = lax.axis_index("s")
    wid = sid * NC + lax.axis_index("c")
    idx_copy = pltpu.make_async_copy(idx_hbm.at[wid, pl.ds(0, BPW * L)],
                                     idx_t, ssem)
    idx_copy.start()
    snum = VP // NS
    soff = sid * snum
    for o, n in ((0, 25600), (25600, 25600), (51200, snum - 51200)):
        pltpu.sync_copy(t_hbm.at[pl.ds(soff + o, n)], vals_v.at[pl.ds(0, n)])
        pltpu.sync_copy(vals_v.at[pl.ds(0, n)], t_sh.at[pl.ds(soff + o, n)])
    idx_copy.wait()
    plsc.subcore_barrier()
    half = (L // 2) * BPW
    g0 = pltpu.make_async_copy(t_sh.at[idx_t.at[pl.ds(0, half)]],
                               vals_v.at[pl.ds(0, half)], sems.at[0])
    g1 = pltpu.make_async_copy(
        t_sh.at[idx_t.at[pl.ds(half, BPW * L - half)]],
        vals_v.at[pl.ds(half, BPW * L - half)], sems.at[1])
    g0.start()
    g1.start()
    for c, (l0, l1) in enumerate(((0, L // 2), (L // 2, L))):
        (g0 if c == 0 else g1).wait()
        for jg in range(JG):
            base = jg * 16
            acc = vals_v[pl.ds(l0 * BPW + base, 16)]
            for l in range(l0 + 1, l1):
                acc = acc + vals_v[pl.ds(l * BPW + base, 16)]
            if c == 0:
                acc_v[pl.ds(base, 16)] = acc
            else:
                acc_v[pl.ds(base, 16)] += acc
    pltpu.sync_copy(acc_v, out_hbm.at[pl.ds(wid * BPW, BPW)])


def kernel(input, input_lengths, table, W):
    del input_lengths
    t = _project_table(table, W)
    idx = _reblock_idx(input.astype(jnp.int32).T).reshape(NW, LP * BPW)
    out = _sc_gather_sum(idx, t)
    return out.reshape(B, 1)


# speedup vs baseline: 1.0131x; 1.0131x over previous
import functools

import jax
import jax.numpy as jnp
from jax import lax
from jax.experimental import pallas as pl
from jax.experimental.pallas import tpu as pltpu
from jax.experimental.pallas import tpu_sc as plsc

B, L, V, D = 16384, 50, 1000000, 32
VP = 1000064

NC, NS = 2, 16
NW = NC * NS
BPW = B // NW
JG = BPW // 16
LP = 56

_ND = D // 8
_CH = 124928
_NCH = 8
_TAIL = V - _NCH * _CH
_CHUNKS = [(k * _CH, _CH) for k in range(_NCH)] + [(_NCH * _CH, _TAIL)]
_NBUF = 4


def _tc_project_body(tv_hbm, w_ref, t_ref, buf, tbuf, sems):
    jobs = [(i, off, n) for i in range(_ND) for (off, n) in _CHUNKS]

    def copy_in(slot, job):
        i, off, n = job
        dst = buf.at[slot] if n == _CH else tbuf.at[slot]
        return pltpu.make_async_copy(
            tv_hbm.at[pl.ds(8 * i, 8), pl.ds(off, n)],
            dst,
            sems.at[slot],
        )

    nbuf = _NBUF
    for p in range(min(nbuf - 1, len(jobs))):
        copy_in(p % nbuf, jobs[p]).start()
    for j, job in enumerate(jobs):
        if j + nbuf - 1 < len(jobs):
            copy_in((j + nbuf - 1) % nbuf, jobs[j + nbuf - 1]).start()
        copy_in(j % nbuf, job).wait()
        i, off, n = job
        src = buf[j % nbuf] if n == _CH else tbuf[j % nbuf]
        part = jnp.sum(src * w_ref[pl.ds(8 * i, 8), :], axis=0)
        if i == 0:
            t_ref[pl.ds(off, n)] = part
        else:
            t_ref[pl.ds(off, n)] += part


def _project_table(table, W):
    tv = table.T
    wcol = W.reshape(D, 1)
    t = pl.pallas_call(
        _tc_project_body,
        in_specs=[
            pl.BlockSpec(memory_space=pl.ANY),
            pl.BlockSpec((D, 1), lambda: (0, 0)),
        ],
        out_specs=pl.BlockSpec((VP,), lambda: (0,)),
        out_shape=jax.ShapeDtypeStruct((VP,), jnp.float32),
        scratch_shapes=[
            pltpu.VMEM((_NBUF, 8, _CH), jnp.float32),
            pltpu.VMEM((_NBUF, 8, _TAIL), jnp.float32),
            pltpu.SemaphoreType.DMA((_NBUF,)),
        ],
    )(tv, wcol)
    return t


def _tc_reblock_body(idx_ref, out_ref):
    out_ref[pl.ds(0, L), :] = idx_ref[...]


def _reblock_idx(idx_t):
    return pl.pallas_call(
        _tc_reblock_body,
        grid=(NW,),
        in_specs=[pl.BlockSpec((L, BPW), lambda i: (0, i))],
        out_specs=pl.BlockSpec((LP, BPW), lambda i: (i, 0)),
        out_shape=jax.ShapeDtypeStruct((NW * LP, BPW), jnp.int32),
    )(idx_t)


@functools.partial(
    pl.kernel,
    out_type=jax.ShapeDtypeStruct((B,), jnp.float32),
    mesh=plsc.VectorSubcoreMesh(core_axis_name="c", subcore_axis_name="s"),
    scratch_types=[
        pltpu.VMEM((BPW * L,), jnp.int32),
        pltpu.VMEM((BPW * L,), jnp.float32),
        pltpu.VMEM((BPW,), jnp.float32),
        pltpu.VMEM_SHARED((VP,), jnp.float32),
        pltpu.SemaphoreType.DMA((2,)),
        pltpu.SemaphoreType.DMA,
    ],
)
def _sc_gather_sum(idx_hbm, t_hbm, out_hbm, idx_t, vals_v, acc_v, t_sh,
                   sems, ssem):
    sid = lax.axis_index("s")
    wid = sid * NC + lax.axis_index("c")
    idx_copy = pltpu.make_async_copy(idx_hbm.at[wid, pl.ds(0, BPW * L)],
                                     idx_t, ssem)
    idx_copy.start()
    snum = VP // NS
    soff = sid * snum
    half = (BPW * L) // 2
    slices = [(k * 12800, min(12800, snum - k * 12800))
              for k in range((snum + 12799) // 12800)]
    hop1 = [pltpu.make_async_copy(
        t_hbm.at[pl.ds(soff + o, n)],
        vals_v.at[pl.ds((k % 2) * half, n)], sems.at[k % 2])
        for k, (o, n) in enumerate(slices)]
    hop2 = [pltpu.make_async_copy(
        vals_v.at[pl.ds((k % 2) * half, n)],
        t_sh.at[pl.ds(soff + o, n)], sems.at[k % 2])
        for k, (o, n) in enumerate(slices)]
    hop1[0].start()
    for k in range(len(slices)):
        hop1[k].wait()
        if k > 0:
            hop2[k - 1].wait()
        if k + 1 < len(slices):
            hop1[k + 1].start()
        hop2[k].start()
    hop2[-1].wait()
    idx_copy.wait()
    plsc.subcore_barrier()
    half = (L // 2) * BPW
    g0 = pltpu.make_async_copy(t_sh.at[idx_t.at[pl.ds(0, half)]],
                               vals_v.at[pl.ds(0, half)], sems.at[0])
    g1 = pltpu.make_async_copy(
        t_sh.at[idx_t.at[pl.ds(half, BPW * L - half)]],
        vals_v.at[pl.ds(half, BPW * L - half)], sems.at[1])
    g0.start()
    g1.start()
    for c, (l0, l1) in enumerate(((0, L // 2), (L // 2, L))):
        (g0 if c == 0 else g1).wait()
        for jg in range(JG):
            base = jg * 16
            acc = vals_v[pl.ds(l0 * BPW + base, 16)]
            for l in range(l0 + 1, l1):
                acc = acc + vals_v[pl.ds(l * BPW + base, 16)]
            if c == 0:
                acc_v[pl.ds(base, 16)] = acc
            else:
                acc_v[pl.ds(base, 16)] += acc
    pltpu.sync_copy(acc_v, out_hbm.at[pl.ds(wid * BPW, BPW)])


def kernel(input, input_lengths, table, W):
    del input_lengths
    t = _project_table(table, W)
    idx = _reblock_idx(input.astype(jnp.int32).T).reshape(NW, LP * BPW)
    out = _sc_gather_sum(idx, t)
    return out.reshape(B, 1)
